# baseline (speedup 1.0000x reference)
import functools

import jax
import jax.numpy as jnp
from jax import lax
from jax.experimental import pallas as pl
from jax.experimental.pallas import tpu as pltpu
from jax.experimental.pallas import tpu_sc as plsc

N = 10000
NPAD = 10240
E = 320000
F = 128
CH = 4
NC = 2
NS = 16
NW = NC * NS
W = 128
WIN = 80
WA = 64
WINA = 160
EPW = W * WIN
EP = NW * EPW
RPT = NPAD // NS
RB = 256
GRID = NPAD // RB
f32 = jnp.float32
i32 = jnp.int32

_mesh = plsc.VectorSubcoreMesh(
    core_axis_name="c", subcore_axis_name="s", num_cores=NC, num_subcores=NS)
_sc_params = pltpu.CompilerParams(use_tc_tiling_on_sc=False)


def _zero_rows(buf, nrows):
    def body(r, _):
        for jc in range(F // 16):
            buf[r, pl.ds(jc * 16, 16)] = jnp.zeros((16,), f32)
        return 0
    lax.fori_loop(0, nrows, body, 0)


def _scale_window(buf, tbuf, wbase, n):
    def chunk(chk, _):
        t16 = tbuf[pl.ds(wbase + chk * 16, 16)]
        for l in range(16):
            e = chk * 16 + l
            t = t16[l]
            for jc in range(F // 16):
                buf[e, pl.ds(jc * 16, 16)] = buf[e, pl.ds(jc * 16, 16)] * t
        return 0
    lax.fori_loop(0, n // 16, chunk, 0)


def _unpack(pw, base, n, colv, rowv):
    def b(i, _):
        p = pw[pl.ds(base + i * 16, 16)]
        if colv is not None:
            colv[pl.ds(i * 16, 16)] = p & 0xFFFF
        rowv[pl.ds(i * 16, 16)] = p >> 16
        return 0
    lax.fori_loop(0, n // 16, b, 0)


@functools.partial(
    pl.kernel,
    out_type=jax.ShapeDtypeStruct((NC, NPAD), f32),
    mesh=_mesh,
    scratch_types=[
        pltpu.VMEM((EPW,), i32),
        pltpu.VMEM((W,), i32),
        pltpu.VMEM((W,), f32),
        pltpu.VMEM((RPT,), f32),
        pltpu.VMEM_SHARED((NPAD,), f32),
    ],
)
def _deg_sc(pk_hbm, out_hbm, pw, rowv, ones_v, zv, acc):
    c = lax.axis_index("c")
    s = lax.axis_index("s")
    wid = c * NS + s
    pltpu.sync_copy(pk_hbm.at[wid], pw)
    for i in range(W // 16):
        ones_v[pl.ds(i * 16, 16)] = jnp.ones((16,), f32)

    def zb(i, _):
        zv[pl.ds(i * 16, 16)] = jnp.zeros((16,), f32)
        return 0
    lax.fori_loop(0, RPT // 16, zb, 0)
    pltpu.sync_copy(zv, acc.at[pl.ds(s * RPT, RPT)])
    plsc.subcore_barrier()

    def win(w, _):
        _unpack(pw, w * W, W, None, rowv)
        pltpu.sync_copy(ones_v, acc.at[rowv], add=True)
        return 0
    lax.fori_loop(0, WIN, win, 0)
    plsc.subcore_barrier()
    pltpu.sync_copy(acc.at[pl.ds(s * RPT, RPT)],
                    out_hbm.at[c, pl.ds(s * RPT, RPT)])


@functools.partial(
    pl.kernel,
    out_type=jax.ShapeDtypeStruct((NC, NPAD, F), f32),
    mesh=_mesh,
    scratch_types=[
        pltpu.VMEM((EPW,), i32),
        pltpu.VMEM((W,), i32),
        pltpu.VMEM((W,), i32),
        pltpu.VMEM((W,), i32),
        pltpu.VMEM((W,), i32),
        pltpu.VMEM((W, F), f32),
        pltpu.VMEM((W, F), f32),
        pltpu.VMEM_SHARED((NPAD, F), f32),
        pltpu.SemaphoreType.DMA,
        pltpu.SemaphoreType.DMA,
        pltpu.SemaphoreType.DMA,
        pltpu.SemaphoreType.DMA,
    ],
)
def _hsum_sc(pk_hbm, xws_hbm, out_hbm, pw, cva, rva, cvb, rvb, bufa, bufb,
             acc, sga, sgb, ssa, ssb):
    c = lax.axis_index("c")
    s = lax.axis_index("s")
    wid = c * NS + s
    pltpu.sync_copy(pk_hbm.at[wid], pw)
    _zero_rows(bufa, W)
    for i in range(RPT // W):
        pltpu.sync_copy(bufa, acc.at[pl.ds(s * RPT + i * W, W)])
    plsc.subcore_barrier()

    _unpack(pw, 0, W, cva, rva)
    pltpu.async_copy(xws_hbm.at[cva], bufa, sga)
    _unpack(pw, W, W, cvb, rvb)
    pltpu.async_copy(xws_hbm.at[cvb], bufb, sgb)

    def win2(i, _):
        w0 = 2 * i
        pltpu.make_async_copy(xws_hbm.at[cva], bufa, sga).wait()
        pltpu.sync_copy(bufa, acc.at[rva], add=True)

        @pl.when(i < WIN // 2 - 1)
        def _():
            _unpack(pw, (w0 + 2) * W, W, cva, rva)
            pltpu.async_copy(xws_hbm.at[cva], bufa, sga)
        pltpu.make_async_copy(xws_hbm.at[cvb], bufb, sgb).wait()
        pltpu.sync_copy(bufb, acc.at[rvb], add=True)

        @pl.when(i < WIN // 2 - 1)
        def _():
            _unpack(pw, (w0 + 3) * W, W, cvb, rvb)
            pltpu.async_copy(xws_hbm.at[cvb], bufb, sgb)
        return 0
    lax.fori_loop(0, WIN // 2, win2, 0)
    plsc.subcore_barrier()
    pltpu.sync_copy(acc.at[pl.ds(s * RPT, RPT)],
                    out_hbm.at[c, pl.ds(s * RPT, RPT)])


@functools.partial(
    pl.kernel,
    out_type=jax.ShapeDtypeStruct((CH, NC, NPAD, F), f32),
    mesh=_mesh,
    scratch_types=[
        pltpu.VMEM((EPW,), i32),
        pltpu.VMEM((WA,), i32),
        pltpu.VMEM((WA,), i32),
        pltpu.VMEM((WA,), i32),
        pltpu.VMEM((WA,), i32),
        pltpu.VMEM((WA, F), f32),
        pltpu.VMEM((WA, F), f32),
        pltpu.VMEM((WA, 16), f32),
        pltpu.VMEM((WA, 16), f32),
        pltpu.VMEM((WA, 16), f32),
        pltpu.VMEM((WA, 16), f32),
        pltpu.VMEM((EPW,), f32),
        pltpu.VMEM_SHARED((NPAD, F), f32),
        pltpu.SemaphoreType.DMA,
        pltpu.SemaphoreType.DMA,
        pltpu.SemaphoreType.DMA,
        pltpu.SemaphoreType.DMA,
    ],
    compiler_params=_sc_params,
)
def _agg_sc(pk_hbm, g_hbm, q_hbm, z0, z1, z2, z3, out_hbm,
            pw, cva, rva, cvb, rvb, bufa, bufb, ga, qa, gb, qb, tbuf, acc,
            sga, sgb, ssa, ssb):
    c = lax.axis_index("c")
    s = lax.axis_index("s")
    wid = c * NS + s
    pltpu.sync_copy(pk_hbm.at[wid], pw)
    lane = jnp.arange(16, dtype=i32)

    def tcompute(gbuf, qbuf, wbase):
        def tchunk(chk, _):
            t16 = jnp.zeros((16,), f32)
            for l in range(16):
                e = chk * 16 + l
                pe = gbuf[e, :] * qbuf[e, :]
                sv = jnp.broadcast_to(pe[0] + pe[1] + pe[2] + pe[3], (16,))
                t16 = jnp.where(lane == l, 1.0 / sv, t16)
            tbuf[pl.ds(wbase + chk * 16, 16)] = t16
            return 0
        lax.fori_loop(0, WA // 16, tchunk, 0)

    _unpack(pw, 0, WA, cva, rva)
    pltpu.async_copy(g_hbm.at[cva], ga, sga)
    pltpu.async_copy(q_hbm.at[rva], qa, ssa)
    _unpack(pw, WA, WA, cvb, rvb)
    pltpu.async_copy(g_hbm.at[cvb], gb, sgb)
    pltpu.async_copy(q_hbm.at[rvb], qb, ssb)

    def twin(i, _):
        w0 = 2 * i
        pltpu.make_async_copy(g_hbm.at[cva], ga, sga).wait()
        pltpu.make_async_copy(q_hbm.at[rva], qa, ssa).wait()
        tcompute(ga, qa, w0 * WA)

        @pl.when(i < WINA // 2 - 1)
        def _():
            _unpack(pw, (w0 + 2) * WA, WA, cva, rva)
            pltpu.async_copy(g_hbm.at[cva], ga, sga)
            pltpu.async_copy(q_hbm.at[rva], qa, ssa)
        pltpu.make_async_copy(g_hbm.at[cvb], gb, sgb).wait()
        pltpu.make_async_copy(q_hbm.at[rvb], qb, ssb).wait()
        tcompute(gb, qb, (w0 + 1) * WA)

        @pl.when(i < WINA // 2 - 1)
        def _():
            _unpack(pw, (w0 + 3) * WA, WA, cvb, rvb)
            pltpu.async_copy(g_hbm.at[cvb], gb, sgb)
            pltpu.async_copy(q_hbm.at[rvb], qb, ssb)
        return 0
    lax.fori_loop(0, WINA // 2, twin, 0)

    for k, z_hbm in enumerate((z0, z1, z2, z3)):
        _zero_rows(bufa, WA)
        for i in range(RPT // WA):
            pltpu.sync_copy(bufa, acc.at[pl.ds(s * RPT + i * WA, WA)])
        plsc.subcore_barrier()

        _unpack(pw, 0, WA, cva, rva)
        pltpu.async_copy(z_hbm.at[cva], bufa, sga)
        _unpack(pw, WA, WA, cvb, rvb)
        pltpu.async_copy(z_hbm.at[cvb], bufb, sgb)

        def win2(i, _):
            w0 = 2 * i
            pltpu.make_async_copy(z_hbm.at[cva], bufa, sga).wait()
            _scale_window(bufa, tbuf, w0 * WA, WA)
            pltpu.async_copy(bufa, acc.at[rva], ssa, add=True)
            pltpu.make_async_copy(z_hbm.at[cvb], bufb, sgb).wait()
            _scale_window(bufb, tbuf, (w0 + 1) * WA, WA)
            pltpu.make_async_copy(bufa, acc.at[rva], ssa).wait()
            pltpu.async_copy(bufb, acc.at[rvb], ssb, add=True)

            @pl.when(i < WINA // 2 - 1)
            def _():
                _unpack(pw, (w0 + 2) * WA, WA, cva, rva)
                pltpu.async_copy(z_hbm.at[cva], bufa, sga)
            pltpu.make_async_copy(bufb, acc.at[rvb], ssb).wait()

            @pl.when(i < WINA // 2 - 1)
            def _():
                _unpack(pw, (w0 + 3) * WA, WA, cvb, rvb)
                pltpu.async_copy(z_hbm.at[cvb], bufb, sgb)
            return 0
        lax.fori_loop(0, WINA // 2, win2, 0)
        plsc.subcore_barrier()
        pltpu.sync_copy(acc.at[pl.ds(s * RPT, RPT)],
                        out_hbm.at[k, c, pl.ds(s * RPT, RPT)])


def _mm1_body(x_ref, w_ref, o_ref):
    o_ref[...] = jnp.dot(x_ref[...], w_ref[...], preferred_element_type=f32)


_mm1 = pl.pallas_call(
    _mm1_body,
    grid=(GRID,),
    in_specs=[
        pl.BlockSpec((RB, F), lambda i: (i, 0)),
        pl.BlockSpec((F, F), lambda i: (0, 0)),
    ],
    out_specs=pl.BlockSpec((RB, F), lambda i: (i, 0)),
    out_shape=jax.ShapeDtypeStruct((NPAD, F), f32),
)


def _scale_body(xw_ref, d0_ref, d1_ref, o_ref):
    deg = 1.0 + d0_ref[...] + d1_ref[...]
    dinv = lax.rsqrt(deg)
    o_ref[...] = xw_ref[...] * dinv


_scale = pl.pallas_call(
    _scale_body,
    grid=(GRID,),
    in_specs=[
        pl.BlockSpec((RB, F), lambda i: (i, 0)),
        pl.BlockSpec((RB, 1), lambda i: (i, 0)),
        pl.BlockSpec((RB, 1), lambda i: (i, 0)),
    ],
    out_specs=pl.BlockSpec((RB, F), lambda i: (i, 0)),
    out_shape=jax.ShapeDtypeStruct((NPAD, F), f32),
)


def _mid_body(hs0_ref, hs1_ref, xws_ref, d0_ref, d1_ref, x_ref, b1_ref,
              wfca_ref, wfcb_ref, bfcp_ref, qb_ref,
              g_ref, q_ref, z0_ref, z1_ref, z2_ref, z3_ref):
    dinv = lax.rsqrt(1.0 + d0_ref[...] + d1_ref[...])
    h = (hs0_ref[...] + hs1_ref[...] + xws_ref[...]) * dinv + b1_ref[...]
    lc = jnp.dot(h, wfca_ref[...], preferred_element_type=f32) + bfcp_ref[...]
    lr = jnp.dot(h, wfcb_ref[...], preferred_element_type=f32) + qb_ref[...]
    g = jnp.exp(lc - jnp.max(lc, axis=1, keepdims=True))
    q = jnp.exp(lr - jnp.max(lr, axis=1, keepdims=True))
    g_ref[...] = g
    q_ref[...] = q
    xb = x_ref[...]
    z0_ref[...] = xb * g[:, 0:1]
    z1_ref[...] = xb * g[:, 1:2]
    z2_ref[...] = xb * g[:, 2:3]
    z3_ref[...] = xb * g[:, 3:4]


_mid = pl.pallas_call(
    _mid_body,
    grid=(GRID,),
    in_specs=[
        pl.BlockSpec((RB, F), lambda i: (i, 0)),
        pl.BlockSpec((RB, F), lambda i: (i, 0)),
        pl.BlockSpec((RB, F), lambda i: (i, 0)),
        pl.BlockSpec((RB, 1), lambda i: (i, 0)),
        pl.BlockSpec((RB, 1), lambda i: (i, 0)),
        pl.BlockSpec((RB, F), lambda i: (i, 0)),
        pl.BlockSpec((1, F), lambda i: (0, 0)),
        pl.BlockSpec((F, 16), lambda i: (0, 0)),
        pl.BlockSpec((F, 16), lambda i: (0, 0)),
        pl.BlockSpec((1, 16), lambda i: (0, 0)),
        pl.BlockSpec((1, 16), lambda i: (0, 0)),
    ],
    out_specs=[
        pl.BlockSpec((RB, 16), lambda i: (i, 0)),
        pl.BlockSpec((RB, 16), lambda i: (i, 0)),
        pl.BlockSpec((RB, F), lambda i: (i, 0)),
        pl.BlockSpec((RB, F), lambda i: (i, 0)),
        pl.BlockSpec((RB, F), lambda i: (i, 0)),
        pl.BlockSpec((RB, F), lambda i: (i, 0)),
    ],
    out_shape=[
        jax.ShapeDtypeStruct((NPAD, 16), f32),
        jax.ShapeDtypeStruct((NPAD, 16), f32),
        jax.ShapeDtypeStruct((NPAD, F), f32),
        jax.ShapeDtypeStruct((NPAD, F), f32),
        jax.ShapeDtypeStruct((NPAD, F), f32),
        jax.ShapeDtypeStruct((NPAD, F), f32),
    ],
)


def _final_body(ag_ref, q_ref, w_ref, b_ref, oa_ref):
    k = pl.program_id(0)
    a = ag_ref[0, 0] + ag_ref[0, 1]
    qall = q_ref[...]
    onehot = lax.broadcasted_iota(i32, (RB, 16), 1) == k
    qk = jnp.sum(jnp.where(onehot, qall, 0.0), axis=1, keepdims=True)
    o = jnp.dot(a * qk, w_ref[0], preferred_element_type=f32) + b_ref[0]
    nrm = jnp.sqrt(jnp.sum(o * o, axis=1, keepdims=True))
    o = o / jnp.maximum(nrm, 1e-12)
    oa_ref[0] = o


_final = pl.pallas_call(
    _final_body,
    grid=(CH, GRID),
    in_specs=[
        pl.BlockSpec((1, NC, RB, F), lambda k, i: (k, 0, i, 0)),
        pl.BlockSpec((RB, 16), lambda k, i: (i, 0)),
        pl.BlockSpec((1, F, F), lambda k, i: (k, 0, 0)),
        pl.BlockSpec((1, 1, F), lambda k, i: (k, 0, 0)),
    ],
    out_specs=pl.BlockSpec((1, RB, F), lambda k, i: (k, i, 0)),
    out_shape=jax.ShapeDtypeStruct((CH, NPAD, F), f32),
)


def kernel(x, edge_index, W1, b1, Wfc, bfc, Wconv, bias_list):
    row = edge_index[0].astype(i32)
    col = edge_index[1].astype(i32)
    padi = (N + (jnp.arange(EP - E, dtype=i32) % (NPAD - N))).astype(i32)
    rowp = jnp.concatenate([row, padi])
    colp = jnp.concatenate([col, padi])
    packed = (colp | (rowp << 16)).reshape(NW, EPW)
    x_p = jnp.pad(x, ((0, NPAD - N), (0, 0)))

    degp = _deg_sc(packed)
    xw = _mm1(x_p, W1)
    d0 = degp[0].reshape(NPAD, 1)
    d1 = degp[1].reshape(NPAD, 1)
    xws = _scale(xw, d0, d1)
    hsump = _hsum_sc(packed, xws)

    wfca = jnp.pad(Wfc[:F], ((0, 0), (0, 12)))
    wfcb = jnp.pad(Wfc[F:], ((0, 0), (0, 12)))
    neg = jnp.full((12,), -1e30, f32)
    bfcp = jnp.concatenate([bfc, neg]).reshape(1, 16)
    qb = jnp.concatenate([jnp.zeros((CH,), f32), neg]).reshape(1, 16)
    g, q, z0, z1, z2, z3 = _mid(
        hsump[0], hsump[1], xws, d0, d1, x_p, b1.reshape(1, F),
        wfca, wfcb, bfcp, qb)

    aggp = _agg_sc(packed, g, q, z0, z1, z2, z3)
    oa = _final(aggp, q, Wconv, bias_list.reshape(CH, 1, F))

    output = oa[:, :N].transpose(1, 0, 2).reshape(N, CH * F)
    xs = oa[:2, :N].reshape(2 * N, F)
    xus = oa[2:, :N].reshape(2 * N, F)
    return (output, xs, xus)


# speedup vs baseline: 1.0752x; 1.0752x over previous
import functools

import jax
import jax.numpy as jnp
from jax import lax
from jax.experimental import pallas as pl
from jax.experimental.pallas import tpu as pltpu
from jax.experimental.pallas import tpu_sc as plsc

N = 10000
NPAD = 10240
E = 320000
F = 128
CH = 4
NC = 2
NS = 16
NW = NC * NS
W = 128
WIN = 80
WA = 80
WINA = 128
EPW = W * WIN
EP = NW * EPW
RPT = NPAD // NS
RB = 256
GRID = NPAD // RB
f32 = jnp.float32
i32 = jnp.int32

_mesh = plsc.VectorSubcoreMesh(
    core_axis_name="c", subcore_axis_name="s", num_cores=NC, num_subcores=NS)
_sc_params = pltpu.CompilerParams(use_tc_tiling_on_sc=False)


def _zero_rows(buf, nrows):
    def body(r, _):
        for jc in range(F // 16):
            buf[r, pl.ds(jc * 16, 16)] = jnp.zeros((16,), f32)
        return 0
    lax.fori_loop(0, nrows, body, 0)


def _scale_window(buf, tbuf, wbase, n):
    def chunk(chk, _):
        t16 = tbuf[pl.ds(wbase + chk * 16, 16)]
        for l in range(16):
            e = chk * 16 + l
            t = t16[l]
            for jc in range(F // 16):
                buf[e, pl.ds(jc * 16, 16)] = buf[e, pl.ds(jc * 16, 16)] * t
        return 0
    lax.fori_loop(0, n // 16, chunk, 0)


def _unpack(pw, base, n, colv, rowv):
    def b(i, _):
        p = pw[pl.ds(base + i * 16, 16)]
        if colv is not None:
            colv[pl.ds(i * 16, 16)] = p & 0xFFFF
        rowv[pl.ds(i * 16, 16)] = p >> 16
        return 0
    lax.fori_loop(0, n // 16, b, 0)


@functools.partial(
    pl.kernel,
    out_type=jax.ShapeDtypeStruct((NC, NPAD), f32),
    mesh=_mesh,
    scratch_types=[
        pltpu.VMEM((EPW,), i32),
        pltpu.VMEM((W,), i32),
        pltpu.VMEM((W,), f32),
        pltpu.VMEM((RPT,), f32),
        pltpu.VMEM_SHARED((NPAD,), f32),
    ],
)
def _deg_sc(pk_hbm, out_hbm, pw, rowv, ones_v, zv, acc):
    c = lax.axis_index("c")
    s = lax.axis_index("s")
    wid = c * NS + s
    pltpu.sync_copy(pk_hbm.at[wid], pw)
    for i in range(W // 16):
        ones_v[pl.ds(i * 16, 16)] = jnp.ones((16,), f32)

    def zb(i, _):
        zv[pl.ds(i * 16, 16)] = jnp.zeros((16,), f32)
        return 0
    lax.fori_loop(0, RPT // 16, zb, 0)
    pltpu.sync_copy(zv, acc.at[pl.ds(s * RPT, RPT)])
    plsc.subcore_barrier()

    def win(w, _):
        _unpack(pw, w * W, W, None, rowv)
        pltpu.sync_copy(ones_v, acc.at[rowv], add=True)
        return 0
    lax.fori_loop(0, WIN, win, 0)
    plsc.subcore_barrier()
    pltpu.sync_copy(acc.at[pl.ds(s * RPT, RPT)],
                    out_hbm.at[c, pl.ds(s * RPT, RPT)])


@functools.partial(
    pl.kernel,
    out_type=jax.ShapeDtypeStruct((NC, NPAD, F), f32),
    mesh=_mesh,
    scratch_types=[
        pltpu.VMEM((EPW,), i32),
        pltpu.VMEM((W,), i32),
        pltpu.VMEM((W,), i32),
        pltpu.VMEM((W,), i32),
        pltpu.VMEM((W,), i32),
        pltpu.VMEM((W, F), f32),
        pltpu.VMEM((W, F), f32),
        pltpu.VMEM_SHARED((NPAD, F), f32),
        pltpu.SemaphoreType.DMA,
        pltpu.SemaphoreType.DMA,
        pltpu.SemaphoreType.DMA,
        pltpu.SemaphoreType.DMA,
    ],
)
def _hsum_sc(pk_hbm, xws_hbm, out_hbm, pw, cva, rva, cvb, rvb, bufa, bufb,
             acc, sga, sgb, ssa, ssb):
    c = lax.axis_index("c")
    s = lax.axis_index("s")
    wid = c * NS + s
    pltpu.sync_copy(pk_hbm.at[wid], pw)
    _zero_rows(bufa, W)
    for i in range(RPT // W):
        pltpu.sync_copy(bufa, acc.at[pl.ds(s * RPT + i * W, W)])
    plsc.subcore_barrier()

    _unpack(pw, 0, W, cva, rva)
    pltpu.async_copy(xws_hbm.at[cva], bufa, sga)
    _unpack(pw, W, W, cvb, rvb)
    pltpu.async_copy(xws_hbm.at[cvb], bufb, sgb)

    def win2(i, _):
        w0 = 2 * i
        pltpu.make_async_copy(xws_hbm.at[cva], bufa, sga).wait()
        pltpu.sync_copy(bufa, acc.at[rva], add=True)

        @pl.when(i < WIN // 2 - 1)
        def _():
            _unpack(pw, (w0 + 2) * W, W, cva, rva)
            pltpu.async_copy(xws_hbm.at[cva], bufa, sga)
        pltpu.make_async_copy(xws_hbm.at[cvb], bufb, sgb).wait()
        pltpu.sync_copy(bufb, acc.at[rvb], add=True)

        @pl.when(i < WIN // 2 - 1)
        def _():
            _unpack(pw, (w0 + 3) * W, W, cvb, rvb)
            pltpu.async_copy(xws_hbm.at[cvb], bufb, sgb)
        return 0
    lax.fori_loop(0, WIN // 2, win2, 0)
    plsc.subcore_barrier()
    pltpu.sync_copy(acc.at[pl.ds(s * RPT, RPT)],
                    out_hbm.at[c, pl.ds(s * RPT, RPT)])


@functools.partial(
    pl.kernel,
    out_type=jax.ShapeDtypeStruct((CH, NC, NPAD, F), f32),
    mesh=_mesh,
    scratch_types=[
        pltpu.VMEM((EPW,), i32),
        pltpu.VMEM((WA,), i32),
        pltpu.VMEM((WA,), i32),
        pltpu.VMEM((WA,), i32),
        pltpu.VMEM((WA,), i32),
        pltpu.VMEM((WA, F), f32),
        pltpu.VMEM((WA, F), f32),
        pltpu.VMEM((WA, 16), f32),
        pltpu.VMEM((WA, 16), f32),
        pltpu.VMEM((WA, 16), f32),
        pltpu.VMEM((WA, 16), f32),
        pltpu.VMEM((EPW,), f32),
        pltpu.VMEM_SHARED((NPAD, F), f32),
        pltpu.SemaphoreType.DMA,
        pltpu.SemaphoreType.DMA,
        pltpu.SemaphoreType.DMA,
        pltpu.SemaphoreType.DMA,
    ],
    compiler_params=_sc_params,
)
def _agg_sc(pk_hbm, g_hbm, q_hbm, z0, z1, z2, z3, out_hbm,
            pw, cva, rva, cvb, rvb, bufa, bufb, ga, qa, gb, qb, tbuf, acc,
            sga, sgb, ssa, ssb):
    c = lax.axis_index("c")
    s = lax.axis_index("s")
    wid = c * NS + s
    pltpu.sync_copy(pk_hbm.at[wid], pw)
    lane = jnp.arange(16, dtype=i32)

    def tcompute(gbuf, qbuf, wbase):
        def tchunk(chk, _):
            t16 = jnp.zeros((16,), f32)
            for l in range(16):
                e = chk * 16 + l
                pe = gbuf[e, :] * qbuf[e, :]
                sv = jnp.broadcast_to(pe[0] + pe[1] + pe[2] + pe[3], (16,))
                t16 = jnp.where(lane == l, 1.0 / sv, t16)
            tbuf[pl.ds(wbase + chk * 16, 16)] = t16
            return 0
        lax.fori_loop(0, WA // 16, tchunk, 0)

    _unpack(pw, 0, WA, cva, rva)
    pltpu.async_copy(g_hbm.at[cva], ga, sga)
    pltpu.async_copy(q_hbm.at[rva], qa, ssa)
    _unpack(pw, WA, WA, cvb, rvb)
    pltpu.async_copy(g_hbm.at[cvb], gb, sgb)
    pltpu.async_copy(q_hbm.at[rvb], qb, ssb)

    def twin(i, _):
        w0 = 2 * i
        pltpu.make_async_copy(g_hbm.at[cva], ga, sga).wait()
        pltpu.make_async_copy(q_hbm.at[rva], qa, ssa).wait()
        tcompute(ga, qa, w0 * WA)

        @pl.when(i < WINA // 2 - 1)
        def _():
            _unpack(pw, (w0 + 2) * WA, WA, cva, rva)
            pltpu.async_copy(g_hbm.at[cva], ga, sga)
            pltpu.async_copy(q_hbm.at[rva], qa, ssa)
        pltpu.make_async_copy(g_hbm.at[cvb], gb, sgb).wait()
        pltpu.make_async_copy(q_hbm.at[rvb], qb, ssb).wait()
        tcompute(gb, qb, (w0 + 1) * WA)

        @pl.when(i < WINA // 2 - 1)
        def _():
            _unpack(pw, (w0 + 3) * WA, WA, cvb, rvb)
            pltpu.async_copy(g_hbm.at[cvb], gb, sgb)
            pltpu.async_copy(q_hbm.at[rvb], qb, ssb)
        return 0
    lax.fori_loop(0, WINA // 2, twin, 0)

    for k, z_hbm in enumerate((z0, z1, z2, z3)):
        _zero_rows(bufa, WA)
        for i in range(RPT // WA):
            pltpu.sync_copy(bufa, acc.at[pl.ds(s * RPT + i * WA, WA)])
        plsc.subcore_barrier()

        _unpack(pw, 0, WA, cva, rva)
        pltpu.async_copy(z_hbm.at[cva], bufa, sga)
        _unpack(pw, WA, WA, cvb, rvb)
        pltpu.async_copy(z_hbm.at[cvb], bufb, sgb)

        def win2(i, _):
            w0 = 2 * i
            pltpu.make_async_copy(z_hbm.at[cva], bufa, sga).wait()
            _scale_window(bufa, tbuf, w0 * WA, WA)
            pltpu.sync_copy(bufa, acc.at[rva], add=True)

            @pl.when(i < WINA // 2 - 1)
            def _():
                _unpack(pw, (w0 + 2) * WA, WA, cva, rva)
                pltpu.async_copy(z_hbm.at[cva], bufa, sga)
            pltpu.make_async_copy(z_hbm.at[cvb], bufb, sgb).wait()
            _scale_window(bufb, tbuf, (w0 + 1) * WA, WA)
            pltpu.sync_copy(bufb, acc.at[rvb], add=True)

            @pl.when(i < WINA // 2 - 1)
            def _():
                _unpack(pw, (w0 + 3) * WA, WA, cvb, rvb)
                pltpu.async_copy(z_hbm.at[cvb], bufb, sgb)
            return 0
        lax.fori_loop(0, WINA // 2, win2, 0)
        plsc.subcore_barrier()
        pltpu.sync_copy(acc.at[pl.ds(s * RPT, RPT)],
                        out_hbm.at[k, c, pl.ds(s * RPT, RPT)])


def _mm1_body(x_ref, w_ref, o_ref):
    o_ref[...] = jnp.dot(x_ref[...], w_ref[...], preferred_element_type=f32)


_mm1 = pl.pallas_call(
    _mm1_body,
    grid=(GRID,),
    in_specs=[
        pl.BlockSpec((RB, F), lambda i: (i, 0)),
        pl.BlockSpec((F, F), lambda i: (0, 0)),
    ],
    out_specs=pl.BlockSpec((RB, F), lambda i: (i, 0)),
    out_shape=jax.ShapeDtypeStruct((NPAD, F), f32),
)


def _scale_body(xw_ref, d0_ref, d1_ref, o_ref):
    deg = 1.0 + d0_ref[...] + d1_ref[...]
    dinv = lax.rsqrt(deg)
    o_ref[...] = xw_ref[...] * dinv


_scale = pl.pallas_call(
    _scale_body,
    grid=(GRID,),
    in_specs=[
        pl.BlockSpec((RB, F), lambda i: (i, 0)),
        pl.BlockSpec((RB, 1), lambda i: (i, 0)),
        pl.BlockSpec((RB, 1), lambda i: (i, 0)),
    ],
    out_specs=pl.BlockSpec((RB, F), lambda i: (i, 0)),
    out_shape=jax.ShapeDtypeStruct((NPAD, F), f32),
)


def _mid_body(hs0_ref, hs1_ref, xws_ref, d0_ref, d1_ref, x_ref, b1_ref,
              wfca_ref, wfcb_ref, bfcp_ref, qb_ref,
              g_ref, q_ref, z0_ref, z1_ref, z2_ref, z3_ref):
    dinv = lax.rsqrt(1.0 + d0_ref[...] + d1_ref[...])
    h = (hs0_ref[...] + hs1_ref[...] + xws_ref[...]) * dinv + b1_ref[...]
    lc = jnp.dot(h, wfca_ref[...], preferred_element_type=f32) + bfcp_ref[...]
    lr = jnp.dot(h, wfcb_ref[...], preferred_element_type=f32) + qb_ref[...]
    g = jnp.exp(lc - jnp.max(lc, axis=1, keepdims=True))
    q = jnp.exp(lr - jnp.max(lr, axis=1, keepdims=True))
    g_ref[...] = g
    q_ref[...] = q
    xb = x_ref[...]
    z0_ref[...] = xb * g[:, 0:1]
    z1_ref[...] = xb * g[:, 1:2]
    z2_ref[...] = xb * g[:, 2:3]
    z3_ref[...] = xb * g[:, 3:4]


_mid = pl.pallas_call(
    _mid_body,
    grid=(GRID,),
    in_specs=[
        pl.BlockSpec((RB, F), lambda i: (i, 0)),
        pl.BlockSpec((RB, F), lambda i: (i, 0)),
        pl.BlockSpec((RB, F), lambda i: (i, 0)),
        pl.BlockSpec((RB, 1), lambda i: (i, 0)),
        pl.BlockSpec((RB, 1), lambda i: (i, 0)),
        pl.BlockSpec((RB, F), lambda i: (i, 0)),
        pl.BlockSpec((1, F), lambda i: (0, 0)),
        pl.BlockSpec((F, 16), lambda i: (0, 0)),
        pl.BlockSpec((F, 16), lambda i: (0, 0)),
        pl.BlockSpec((1, 16), lambda i: (0, 0)),
        pl.BlockSpec((1, 16), lambda i: (0, 0)),
    ],
    out_specs=[
        pl.BlockSpec((RB, 16), lambda i: (i, 0)),
        pl.BlockSpec((RB, 16), lambda i: (i, 0)),
        pl.BlockSpec((RB, F), lambda i: (i, 0)),
        pl.BlockSpec((RB, F), lambda i: (i, 0)),
        pl.BlockSpec((RB, F), lambda i: (i, 0)),
        pl.BlockSpec((RB, F), lambda i: (i, 0)),
    ],
    out_shape=[
        jax.ShapeDtypeStruct((NPAD, 16), f32),
        jax.ShapeDtypeStruct((NPAD, 16), f32),
        jax.ShapeDtypeStruct((NPAD, F), f32),
        jax.ShapeDtypeStruct((NPAD, F), f32),
        jax.ShapeDtypeStruct((NPAD, F), f32),
        jax.ShapeDtypeStruct((NPAD, F), f32),
    ],
)


def _final_body(ag_ref, q_ref, w_ref, b_ref, oa_ref):
    k = pl.program_id(0)
    a = ag_ref[0, 0] + ag_ref[0, 1]
    qall = q_ref[...]
    onehot = lax.broadcasted_iota(i32, (RB, 16), 1) == k
    qk = jnp.sum(jnp.where(onehot, qall, 0.0), axis=1, keepdims=True)
    o = jnp.dot(a * qk, w_ref[0], preferred_element_type=f32) + b_ref[0]
    nrm = jnp.sqrt(jnp.sum(o * o, axis=1, keepdims=True))
    o = o / jnp.maximum(nrm, 1e-12)
    oa_ref[0] = o


_final = pl.pallas_call(
    _final_body,
    grid=(CH, GRID),
    in_specs=[
        pl.BlockSpec((1, NC, RB, F), lambda k, i: (k, 0, i, 0)),
        pl.BlockSpec((RB, 16), lambda k, i: (i, 0)),
        pl.BlockSpec((1, F, F), lambda k, i: (k, 0, 0)),
        pl.BlockSpec((1, 1, F), lambda k, i: (k, 0, 0)),
    ],
    out_specs=pl.BlockSpec((1, RB, F), lambda k, i: (k, i, 0)),
    out_shape=jax.ShapeDtypeStruct((CH, NPAD, F), f32),
)


def kernel(x, edge_index, W1, b1, Wfc, bfc, Wconv, bias_list):
    row = edge_index[0].astype(i32)
    col = edge_index[1].astype(i32)
    padi = (N + (jnp.arange(EP - E, dtype=i32) % (NPAD - N))).astype(i32)
    rowp = jnp.concatenate([row, padi])
    colp = jnp.concatenate([col, padi])
    packed = (colp | (rowp << 16)).reshape(NW, EPW)
    x_p = jnp.pad(x, ((0, NPAD - N), (0, 0)))

    degp = _deg_sc(packed)
    xw = _mm1(x_p, W1)
    d0 = degp[0].reshape(NPAD, 1)
    d1 = degp[1].reshape(NPAD, 1)
    xws = _scale(xw, d0, d1)
    hsump = _hsum_sc(packed, xws)

    wfca = jnp.pad(Wfc[:F], ((0, 0), (0, 12)))
    wfcb = jnp.pad(Wfc[F:], ((0, 0), (0, 12)))
    neg = jnp.full((12,), -1e30, f32)
    bfcp = jnp.concatenate([bfc, neg]).reshape(1, 16)
    qb = jnp.concatenate([jnp.zeros((CH,), f32), neg]).reshape(1, 16)
    g, q, z0, z1, z2, z3 = _mid(
        hsump[0], hsump[1], xws, d0, d1, x_p, b1.reshape(1, F),
        wfca, wfcb, bfcp, qb)

    aggp = _agg_sc(packed, g, q, z0, z1, z2, z3)
    oa = _final(aggp, q, Wconv, bias_list.reshape(CH, 1, F))

    output = oa[:, :N].transpose(1, 0, 2).reshape(N, CH * F)
    xs = oa[:2, :N].reshape(2 * N, F)
    xus = oa[2:, :N].reshape(2 * N, F)
    return (output, xs, xus)


# speedup vs baseline: 1.0883x; 1.0121x over previous
import functools

import jax
import jax.numpy as jnp
from jax import lax
from jax.experimental import pallas as pl
from jax.experimental.pallas import tpu as pltpu
from jax.experimental.pallas import tpu_sc as plsc

N = 10000
NPAD = 10240
E = 320000
F = 128
CH = 4
NC = 2
NS = 16
NW = NC * NS
W = 128
WIN = 80
WA = 80
WINA = 128
EPW = W * WIN
EP = NW * EPW
RPT = NPAD // NS
RB = 256
GRID = NPAD // RB
f32 = jnp.float32
i32 = jnp.int32

_mesh = plsc.VectorSubcoreMesh(
    core_axis_name="c", subcore_axis_name="s", num_cores=NC, num_subcores=NS)
_sc_params = pltpu.CompilerParams(use_tc_tiling_on_sc=False)


def _zero_rows(buf, nrows):
    def body(r, _):
        for jc in range(F // 16):
            buf[r, pl.ds(jc * 16, 16)] = jnp.zeros((16,), f32)
        return 0
    lax.fori_loop(0, nrows, body, 0)


def _scale_window(buf, tbuf, wbase, n):
    def chunk(chk, _):
        t16 = tbuf[pl.ds(wbase + chk * 16, 16)]
        for l in range(16):
            e = chk * 16 + l
            t = t16[l]
            for jc in range(F // 16):
                buf[e, pl.ds(jc * 16, 16)] = buf[e, pl.ds(jc * 16, 16)] * t
        return 0
    lax.fori_loop(0, n // 16, chunk, 0)


def _unpack(pw, base, n, colv, rowv):
    def b(i, _):
        p = pw[pl.ds(base + i * 16, 16)]
        if colv is not None:
            colv[pl.ds(i * 16, 16)] = p & 0xFFFF
        rowv[pl.ds(i * 16, 16)] = p >> 16
        return 0
    lax.fori_loop(0, n // 16, b, 0)


@functools.partial(
    pl.kernel,
    out_type=jax.ShapeDtypeStruct((NC, NPAD), f32),
    mesh=_mesh,
    scratch_types=[
        pltpu.VMEM((EPW,), i32),
        pltpu.VMEM((W,), i32),
        pltpu.VMEM((W,), f32),
        pltpu.VMEM((RPT,), f32),
        pltpu.VMEM_SHARED((NPAD,), f32),
    ],
)
def _deg_sc(pk_hbm, out_hbm, pw, rowv, ones_v, zv, acc):
    c = lax.axis_index("c")
    s = lax.axis_index("s")
    wid = c * NS + s
    pltpu.sync_copy(pk_hbm.at[wid], pw)
    for i in range(W // 16):
        ones_v[pl.ds(i * 16, 16)] = jnp.ones((16,), f32)

    def zb(i, _):
        zv[pl.ds(i * 16, 16)] = jnp.zeros((16,), f32)
        return 0
    lax.fori_loop(0, RPT // 16, zb, 0)
    pltpu.sync_copy(zv, acc.at[pl.ds(s * RPT, RPT)])
    plsc.subcore_barrier()

    def win(w, _):
        _unpack(pw, w * W, W, None, rowv)
        pltpu.sync_copy(ones_v, acc.at[rowv], add=True)
        return 0
    lax.fori_loop(0, WIN, win, 0)
    plsc.subcore_barrier()
    pltpu.sync_copy(acc.at[pl.ds(s * RPT, RPT)],
                    out_hbm.at[c, pl.ds(s * RPT, RPT)])


@functools.partial(
    pl.kernel,
    out_type=jax.ShapeDtypeStruct((NC, NPAD, F), f32),
    mesh=_mesh,
    scratch_types=[
        pltpu.VMEM((EPW,), i32),
        pltpu.VMEM((W,), i32),
        pltpu.VMEM((W,), i32),
        pltpu.VMEM((W,), i32),
        pltpu.VMEM((W,), i32),
        pltpu.VMEM((W, F), f32),
        pltpu.VMEM((W, F), f32),
        pltpu.VMEM_SHARED((NPAD, F), f32),
        pltpu.SemaphoreType.DMA,
        pltpu.SemaphoreType.DMA,
        pltpu.SemaphoreType.DMA,
        pltpu.SemaphoreType.DMA,
    ],
)
def _hsum_sc(pk_hbm, xws_hbm, out_hbm, pw, cva, rva, cvb, rvb, bufa, bufb,
             acc, sga, sgb, ssa, ssb):
    c = lax.axis_index("c")
    s = lax.axis_index("s")
    wid = c * NS + s
    pltpu.sync_copy(pk_hbm.at[wid], pw)
    _zero_rows(bufa, W)
    for i in range(RPT // W):
        pltpu.sync_copy(bufa, acc.at[pl.ds(s * RPT + i * W, W)])
    plsc.subcore_barrier()

    _unpack(pw, 0, W, cva, rva)
    pltpu.async_copy(xws_hbm.at[cva], bufa, sga)
    _unpack(pw, W, W, cvb, rvb)
    pltpu.async_copy(xws_hbm.at[cvb], bufb, sgb)

    def win2(i, _):
        w0 = 2 * i
        pltpu.make_async_copy(xws_hbm.at[cva], bufa, sga).wait()
        pltpu.sync_copy(bufa, acc.at[rva], add=True)

        @pl.when(i < WIN // 2 - 1)
        def _():
            _unpack(pw, (w0 + 2) * W, W, cva, rva)
            pltpu.async_copy(xws_hbm.at[cva], bufa, sga)
        pltpu.make_async_copy(xws_hbm.at[cvb], bufb, sgb).wait()
        pltpu.sync_copy(bufb, acc.at[rvb], add=True)

        @pl.when(i < WIN // 2 - 1)
        def _():
            _unpack(pw, (w0 + 3) * W, W, cvb, rvb)
            pltpu.async_copy(xws_hbm.at[cvb], bufb, sgb)
        return 0
    lax.fori_loop(0, WIN // 2, win2, 0)
    plsc.subcore_barrier()
    pltpu.sync_copy(acc.at[pl.ds(s * RPT, RPT)],
                    out_hbm.at[c, pl.ds(s * RPT, RPT)])


@functools.partial(
    pl.kernel,
    out_type=jax.ShapeDtypeStruct((CH, NC, NPAD, F), f32),
    mesh=_mesh,
    scratch_types=[
        pltpu.VMEM((EPW,), i32),
        pltpu.VMEM((WA,), i32),
        pltpu.VMEM((WA,), i32),
        pltpu.VMEM((WA,), i32),
        pltpu.VMEM((WA,), i32),
        pltpu.VMEM((WA, F), f32),
        pltpu.VMEM((WA, F), f32),
        pltpu.VMEM((WA, 16), f32),
        pltpu.VMEM((WA, 16), f32),
        pltpu.VMEM((WA, 16), f32),
        pltpu.VMEM((WA, 16), f32),
        pltpu.VMEM((EPW,), f32),
        pltpu.VMEM_SHARED((NPAD, F), f32),
        pltpu.SemaphoreType.DMA,
        pltpu.SemaphoreType.DMA,
        pltpu.SemaphoreType.DMA,
        pltpu.SemaphoreType.DMA,
    ],
    compiler_params=_sc_params,
)
def _agg_sc(pk_hbm, g_hbm, q_hbm, z0, z1, z2, z3, out_hbm,
            pw, cva, rva, cvb, rvb, bufa, bufb, ga, qa, gb, qb, tbuf, acc,
            sga, sgb, ssa, ssb):
    c = lax.axis_index("c")
    s = lax.axis_index("s")
    wid = c * NS + s
    pltpu.sync_copy(pk_hbm.at[wid], pw)
    lane = jnp.arange(16, dtype=i32)

    def tcompute(gbuf, qbuf, wbase):
        def tchunk(chk, _):
            t16 = jnp.zeros((16,), f32)
            for l in range(16):
                e = chk * 16 + l
                pe = gbuf[e, :] * qbuf[e, :]
                sv = jnp.broadcast_to(pe[0] + pe[1] + pe[2] + pe[3], (16,))
                t16 = jnp.where(lane == l, 1.0 / sv, t16)
            tbuf[pl.ds(wbase + chk * 16, 16)] = t16
            return 0
        lax.fori_loop(0, WA // 16, tchunk, 0)

    _unpack(pw, 0, WA, cva, rva)
    pltpu.async_copy(g_hbm.at[cva], ga, sga)
    pltpu.async_copy(q_hbm.at[rva], qa, ssa)
    _unpack(pw, WA, WA, cvb, rvb)
    pltpu.async_copy(g_hbm.at[cvb], gb, sgb)
    pltpu.async_copy(q_hbm.at[rvb], qb, ssb)

    def twin(i, _):
        w0 = 2 * i
        pltpu.make_async_copy(g_hbm.at[cva], ga, sga).wait()
        pltpu.make_async_copy(q_hbm.at[rva], qa, ssa).wait()
        tcompute(ga, qa, w0 * WA)

        @pl.when(i < WINA // 2 - 1)
        def _():
            _unpack(pw, (w0 + 2) * WA, WA, cva, rva)
            pltpu.async_copy(g_hbm.at[cva], ga, sga)
            pltpu.async_copy(q_hbm.at[rva], qa, ssa)
        pltpu.make_async_copy(g_hbm.at[cvb], gb, sgb).wait()
        pltpu.make_async_copy(q_hbm.at[rvb], qb, ssb).wait()
        tcompute(gb, qb, (w0 + 1) * WA)

        @pl.when(i < WINA // 2 - 1)
        def _():
            _unpack(pw, (w0 + 3) * WA, WA, cvb, rvb)
            pltpu.async_copy(g_hbm.at[cvb], gb, sgb)
            pltpu.async_copy(q_hbm.at[rvb], qb, ssb)
        return 0
    lax.fori_loop(0, WINA // 2, twin, 0)

    for k, z_hbm in enumerate((z0, z1, z2, z3)):
        _zero_rows(bufa, WA)
        for i in range(RPT // WA):
            pltpu.sync_copy(bufa, acc.at[pl.ds(s * RPT + i * WA, WA)])
        plsc.subcore_barrier()

        _unpack(pw, 0, WA, cva, rva)
        pltpu.async_copy(z_hbm.at[cva], bufa, sga)
        _unpack(pw, WA, WA, cvb, rvb)
        pltpu.async_copy(z_hbm.at[cvb], bufb, sgb)

        def win2(i, _):
            w0 = 2 * i
            pltpu.make_async_copy(z_hbm.at[cva], bufa, sga).wait()
            _scale_window(bufa, tbuf, w0 * WA, WA)
            pltpu.sync_copy(bufa, acc.at[rva], add=True)

            @pl.when(i < WINA // 2 - 1)
            def _():
                _unpack(pw, (w0 + 2) * WA, WA, cva, rva)
                pltpu.async_copy(z_hbm.at[cva], bufa, sga)
            pltpu.make_async_copy(z_hbm.at[cvb], bufb, sgb).wait()
            _scale_window(bufb, tbuf, (w0 + 1) * WA, WA)
            pltpu.sync_copy(bufb, acc.at[rvb], add=True)

            @pl.when(i < WINA // 2 - 1)
            def _():
                _unpack(pw, (w0 + 3) * WA, WA, cvb, rvb)
                pltpu.async_copy(z_hbm.at[cvb], bufb, sgb)
            return 0
        lax.fori_loop(0, WINA // 2, win2, 0)
        plsc.subcore_barrier()
        pltpu.sync_copy(acc.at[pl.ds(s * RPT, RPT)],
                        out_hbm.at[k, c, pl.ds(s * RPT, RPT)])


def _scale_body(x_ref, w_ref, d0_ref, d1_ref, o_ref):
    xw = jnp.dot(x_ref[...], w_ref[...], preferred_element_type=f32)
    dinv = lax.rsqrt(1.0 + d0_ref[...] + d1_ref[...])
    o_ref[...] = xw * dinv


_scale = pl.pallas_call(
    _scale_body,
    grid=(GRID,),
    in_specs=[
        pl.BlockSpec((RB, F), lambda i: (i, 0)),
        pl.BlockSpec((F, F), lambda i: (0, 0)),
        pl.BlockSpec((RB, 1), lambda i: (i, 0)),
        pl.BlockSpec((RB, 1), lambda i: (i, 0)),
    ],
    out_specs=pl.BlockSpec((RB, F), lambda i: (i, 0)),
    out_shape=jax.ShapeDtypeStruct((NPAD, F), f32),
)


def _mid_body(hs0_ref, hs1_ref, xws_ref, d0_ref, d1_ref, x_ref, b1_ref,
              wfca_ref, wfcb_ref, bfcp_ref, qb_ref,
              g_ref, q_ref, z0_ref, z1_ref, z2_ref, z3_ref):
    dinv = lax.rsqrt(1.0 + d0_ref[...] + d1_ref[...])
    h = (hs0_ref[...] + hs1_ref[...] + xws_ref[...]) * dinv + b1_ref[...]
    lc = jnp.dot(h, wfca_ref[...], preferred_element_type=f32) + bfcp_ref[...]
    lr = jnp.dot(h, wfcb_ref[...], preferred_element_type=f32) + qb_ref[...]
    g = jnp.exp(lc - jnp.max(lc, axis=1, keepdims=True))
    q = jnp.exp(lr - jnp.max(lr, axis=1, keepdims=True))
    g_ref[...] = g
    q_ref[...] = q
    xb = x_ref[...]
    z0_ref[...] = xb * g[:, 0:1]
    z1_ref[...] = xb * g[:, 1:2]
    z2_ref[...] = xb * g[:, 2:3]
    z3_ref[...] = xb * g[:, 3:4]


_mid = pl.pallas_call(
    _mid_body,
    grid=(GRID,),
    in_specs=[
        pl.BlockSpec((RB, F), lambda i: (i, 0)),
        pl.BlockSpec((RB, F), lambda i: (i, 0)),
        pl.BlockSpec((RB, F), lambda i: (i, 0)),
        pl.BlockSpec((RB, 1), lambda i: (i, 0)),
        pl.BlockSpec((RB, 1), lambda i: (i, 0)),
        pl.BlockSpec((RB, F), lambda i: (i, 0)),
        pl.BlockSpec((1, F), lambda i: (0, 0)),
        pl.BlockSpec((F, 16), lambda i: (0, 0)),
        pl.BlockSpec((F, 16), lambda i: (0, 0)),
        pl.BlockSpec((1, 16), lambda i: (0, 0)),
        pl.BlockSpec((1, 16), lambda i: (0, 0)),
    ],
    out_specs=[
        pl.BlockSpec((RB, 16), lambda i: (i, 0)),
        pl.BlockSpec((RB, 16), lambda i: (i, 0)),
        pl.BlockSpec((RB, F), lambda i: (i, 0)),
        pl.BlockSpec((RB, F), lambda i: (i, 0)),
        pl.BlockSpec((RB, F), lambda i: (i, 0)),
        pl.BlockSpec((RB, F), lambda i: (i, 0)),
    ],
    out_shape=[
        jax.ShapeDtypeStruct((NPAD, 16), f32),
        jax.ShapeDtypeStruct((NPAD, 16), f32),
        jax.ShapeDtypeStruct((NPAD, F), f32),
        jax.ShapeDtypeStruct((NPAD, F), f32),
        jax.ShapeDtypeStruct((NPAD, F), f32),
        jax.ShapeDtypeStruct((NPAD, F), f32),
    ],
)


def _final_body(ag_ref, q_ref, w_ref, b_ref, oa_ref):
    k = pl.program_id(0)
    a = ag_ref[0, 0] + ag_ref[0, 1]
    qall = q_ref[...]
    onehot = lax.broadcasted_iota(i32, (RB, 16), 1) == k
    qk = jnp.sum(jnp.where(onehot, qall, 0.0), axis=1, keepdims=True)
    o = jnp.dot(a * qk, w_ref[0], preferred_element_type=f32) + b_ref[0]
    nrm = jnp.sqrt(jnp.sum(o * o, axis=1, keepdims=True))
    o = o / jnp.maximum(nrm, 1e-12)
    oa_ref[0] = o


_final = pl.pallas_call(
    _final_body,
    grid=(CH, GRID),
    in_specs=[
        pl.BlockSpec((1, NC, RB, F), lambda k, i: (k, 0, i, 0)),
        pl.BlockSpec((RB, 16), lambda k, i: (i, 0)),
        pl.BlockSpec((1, F, F), lambda k, i: (k, 0, 0)),
        pl.BlockSpec((1, 1, F), lambda k, i: (k, 0, 0)),
    ],
    out_specs=pl.BlockSpec((1, RB, F), lambda k, i: (k, i, 0)),
    out_shape=jax.ShapeDtypeStruct((CH, NPAD, F), f32),
)


def kernel(x, edge_index, W1, b1, Wfc, bfc, Wconv, bias_list):
    row = edge_index[0].astype(i32)
    col = edge_index[1].astype(i32)
    padi = (N + (jnp.arange(EP - E, dtype=i32) % (NPAD - N))).astype(i32)
    rowp = jnp.concatenate([row, padi])
    colp = jnp.concatenate([col, padi])
    packed = (colp | (rowp << 16)).reshape(NW, EPW)
    x_p = jnp.pad(x, ((0, NPAD - N), (0, 0)))

    degp = _deg_sc(packed)
    d0 = degp[0].reshape(NPAD, 1)
    d1 = degp[1].reshape(NPAD, 1)
    xws = _scale(x_p, W1, d0, d1)
    hsump = _hsum_sc(packed, xws)

    wfca = jnp.pad(Wfc[:F], ((0, 0), (0, 12)))
    wfcb = jnp.pad(Wfc[F:], ((0, 0), (0, 12)))
    neg = jnp.full((12,), -1e30, f32)
    bfcp = jnp.concatenate([bfc, neg]).reshape(1, 16)
    qb = jnp.concatenate([jnp.zeros((CH,), f32), neg]).reshape(1, 16)
    g, q, z0, z1, z2, z3 = _mid(
        hsump[0], hsump[1], xws, d0, d1, x_p, b1.reshape(1, F),
        wfca, wfcb, bfcp, qb)

    aggp = _agg_sc(packed, g, q, z0, z1, z2, z3)
    oa = _final(aggp, q, Wconv, bias_list.reshape(CH, 1, F))

    output = oa[:, :N].transpose(1, 0, 2).reshape(N, CH * F)
    xs = oa[:2, :N].reshape(2 * N, F)
    xus = oa[2:, :N].reshape(2 * N, F)
    return (output, xs, xus)


# speedup vs baseline: 1.1301x; 1.0385x over previous
import functools

import jax
import jax.numpy as jnp
from jax import lax
from jax.experimental import pallas as pl
from jax.experimental.pallas import tpu as pltpu
from jax.experimental.pallas import tpu_sc as plsc

N = 10000
NPAD = 10240
E = 320000
F = 128
CH = 4
NC = 2
NS = 16
NW = NC * NS
W = 128
WIN = 80
WA = 80
WINA = 128
EPW = W * WIN
EP = NW * EPW
RPT = NPAD // NS
RB = 256
GRID = NPAD // RB
f32 = jnp.float32
i32 = jnp.int32

_mesh = plsc.VectorSubcoreMesh(
    core_axis_name="c", subcore_axis_name="s", num_cores=NC, num_subcores=NS)
_sc_params = pltpu.CompilerParams(use_tc_tiling_on_sc=False)


def _zero_rows(buf, nrows):
    def body(r, _):
        for jc in range(F // 16):
            buf[r, pl.ds(jc * 16, 16)] = jnp.zeros((16,), f32)
        return 0
    lax.fori_loop(0, nrows, body, 0)


def _scale_window(buf, tbuf, wbase, n):
    def chunk(chk, _):
        t16 = tbuf[pl.ds(wbase + chk * 16, 16)]
        for l in range(16):
            e = chk * 16 + l
            t = t16[l]
            for jc in range(F // 16):
                buf[e, pl.ds(jc * 16, 16)] = buf[e, pl.ds(jc * 16, 16)] * t
        return 0
    lax.fori_loop(0, n // 16, chunk, 0)


def _unpack(pw, base, n, colv, rowv):
    def b(i, _):
        p = pw[pl.ds(base + i * 16, 16)]
        if colv is not None:
            colv[pl.ds(i * 16, 16)] = p & 0xFFFF
        rowv[pl.ds(i * 16, 16)] = p >> 16
        return 0
    lax.fori_loop(0, n // 16, b, 0)


@functools.partial(
    pl.kernel,
    out_type=jax.ShapeDtypeStruct((NC, NPAD), f32),
    mesh=_mesh,
    scratch_types=[
        pltpu.VMEM((EPW,), i32),
        pltpu.VMEM((W,), i32),
        pltpu.VMEM((W,), f32),
        pltpu.VMEM((RPT,), f32),
        pltpu.VMEM_SHARED((NPAD,), f32),
    ],
)
def _deg_sc(pk_hbm, out_hbm, pw, rowv, ones_v, zv, acc):
    c = lax.axis_index("c")
    s = lax.axis_index("s")
    wid = c * NS + s
    pltpu.sync_copy(pk_hbm.at[wid], pw)
    for i in range(W // 16):
        ones_v[pl.ds(i * 16, 16)] = jnp.ones((16,), f32)

    def zb(i, _):
        zv[pl.ds(i * 16, 16)] = jnp.zeros((16,), f32)
        return 0
    lax.fori_loop(0, RPT // 16, zb, 0)
    pltpu.sync_copy(zv, acc.at[pl.ds(s * RPT, RPT)])
    plsc.subcore_barrier()

    def win(w, _):
        _unpack(pw, w * W, W, None, rowv)
        pltpu.sync_copy(ones_v, acc.at[rowv], add=True)
        return 0
    lax.fori_loop(0, WIN, win, 0)
    plsc.subcore_barrier()
    pltpu.sync_copy(acc.at[pl.ds(s * RPT, RPT)],
                    out_hbm.at[c, pl.ds(s * RPT, RPT)])


@functools.partial(
    pl.kernel,
    out_type=jax.ShapeDtypeStruct((NC, NPAD, F), f32),
    mesh=_mesh,
    scratch_types=[
        pltpu.VMEM((EPW,), i32),
        pltpu.VMEM((W,), i32),
        pltpu.VMEM((W,), i32),
        pltpu.VMEM((W,), i32),
        pltpu.VMEM((W,), i32),
        pltpu.VMEM((W, F), f32),
        pltpu.VMEM((W, F), f32),
        pltpu.VMEM_SHARED((NPAD, F), f32),
        pltpu.SemaphoreType.DMA,
        pltpu.SemaphoreType.DMA,
        pltpu.SemaphoreType.DMA,
        pltpu.SemaphoreType.DMA,
    ],
)
def _hsum_sc(pk_hbm, xws_hbm, out_hbm, pw, cva, rva, cvb, rvb, bufa, bufb,
             acc, sga, sgb, ssa, ssb):
    c = lax.axis_index("c")
    s = lax.axis_index("s")
    wid = c * NS + s
    pltpu.sync_copy(pk_hbm.at[wid], pw)
    _zero_rows(bufa, W)
    for i in range(RPT // W):
        pltpu.sync_copy(bufa, acc.at[pl.ds(s * RPT + i * W, W)])
    plsc.subcore_barrier()

    _unpack(pw, 0, W, cva, rva)
    pltpu.async_copy(xws_hbm.at[cva], bufa, sga)
    _unpack(pw, W, W, cvb, rvb)
    pltpu.async_copy(xws_hbm.at[cvb], bufb, sgb)

    def win2(i, _):
        w0 = 2 * i
        pltpu.make_async_copy(xws_hbm.at[cva], bufa, sga).wait()
        pltpu.sync_copy(bufa, acc.at[rva], add=True)

        @pl.when(i < WIN // 2 - 1)
        def _():
            _unpack(pw, (w0 + 2) * W, W, cva, rva)
            pltpu.async_copy(xws_hbm.at[cva], bufa, sga)
        pltpu.make_async_copy(xws_hbm.at[cvb], bufb, sgb).wait()
        pltpu.sync_copy(bufb, acc.at[rvb], add=True)

        @pl.when(i < WIN // 2 - 1)
        def _():
            _unpack(pw, (w0 + 3) * W, W, cvb, rvb)
            pltpu.async_copy(xws_hbm.at[cvb], bufb, sgb)
        return 0
    lax.fori_loop(0, WIN // 2, win2, 0)
    plsc.subcore_barrier()
    pltpu.sync_copy(acc.at[pl.ds(s * RPT, RPT)],
                    out_hbm.at[c, pl.ds(s * RPT, RPT)])


@functools.partial(
    pl.kernel,
    out_type=jax.ShapeDtypeStruct((CH, NC, NPAD, F), f32),
    mesh=_mesh,
    scratch_types=[
        pltpu.VMEM((EPW,), i32),
        pltpu.VMEM((WA,), i32),
        pltpu.VMEM((WA,), i32),
        pltpu.VMEM((WA,), i32),
        pltpu.VMEM((WA,), i32),
        pltpu.VMEM((WA, F), f32),
        pltpu.VMEM((WA, F), f32),
        pltpu.VMEM((WA, 16), f32),
        pltpu.VMEM((WA, 16), f32),
        pltpu.VMEM((WA, 16), f32),
        pltpu.VMEM((WA, 16), f32),
        pltpu.VMEM((EPW,), f32),
        pltpu.VMEM_SHARED((NPAD, F), f32),
        pltpu.SemaphoreType.DMA,
        pltpu.SemaphoreType.DMA,
        pltpu.SemaphoreType.DMA,
        pltpu.SemaphoreType.DMA,
    ],
    compiler_params=_sc_params,
)
def _agg_sc(pk_hbm, g_hbm, q_hbm, z0, z1, z2, z3, out_hbm,
            pw, cva, rva, cvb, rvb, bufa, bufb, ga, qa, gb, qb, tbuf, acc,
            sga, sgb, ssa, ssb):
    c = lax.axis_index("c")
    s = lax.axis_index("s")
    wid = c * NS + s
    pltpu.sync_copy(pk_hbm.at[wid], pw)
    lane = jnp.arange(16, dtype=i32)

    def tcompute(gbuf, qbuf, wbase):
        def tchunk(chk, _):
            t16 = jnp.zeros((16,), f32)
            for l in range(16):
                e = chk * 16 + l
                pe = gbuf[e, :] * qbuf[e, :]
                sv = jnp.broadcast_to(pe[0] + pe[1] + pe[2] + pe[3], (16,))
                t16 = jnp.where(lane == l, 1.0 / sv, t16)
            tbuf[pl.ds(wbase + chk * 16, 16)] = t16
            return 0
        lax.fori_loop(0, WA // 16, tchunk, 0)

    for k, z_hbm in enumerate((z0, z1, z2, z3)):
        _zero_rows(bufa, WA)
        for i in range(RPT // WA):
            pltpu.sync_copy(bufa, acc.at[pl.ds(s * RPT + i * WA, WA)])
        plsc.subcore_barrier()

        _unpack(pw, 0, WA, cva, rva)
        pltpu.async_copy(z_hbm.at[cva], bufa, sga)
        if k == 0:
            pltpu.async_copy(g_hbm.at[cva], ga, ssa)
            pltpu.async_copy(q_hbm.at[rva], qa, ssa)
        _unpack(pw, WA, WA, cvb, rvb)
        pltpu.async_copy(z_hbm.at[cvb], bufb, sgb)
        if k == 0:
            pltpu.async_copy(g_hbm.at[cvb], gb, ssb)
            pltpu.async_copy(q_hbm.at[rvb], qb, ssb)

        def win2(i, _):
            w0 = 2 * i
            pltpu.make_async_copy(z_hbm.at[cva], bufa, sga).wait()
            if k == 0:
                pltpu.make_async_copy(g_hbm.at[cva], ga, ssa).wait()
                pltpu.make_async_copy(q_hbm.at[rva], qa, ssa).wait()
                tcompute(ga, qa, w0 * WA)
            _scale_window(bufa, tbuf, w0 * WA, WA)
            pltpu.sync_copy(bufa, acc.at[rva], add=True)

            @pl.when(i < WINA // 2 - 1)
            def _():
                _unpack(pw, (w0 + 2) * WA, WA, cva, rva)
                pltpu.async_copy(z_hbm.at[cva], bufa, sga)
                if k == 0:
                    pltpu.async_copy(g_hbm.at[cva], ga, ssa)
                    pltpu.async_copy(q_hbm.at[rva], qa, ssa)
            pltpu.make_async_copy(z_hbm.at[cvb], bufb, sgb).wait()
            if k == 0:
                pltpu.make_async_copy(g_hbm.at[cvb], gb, ssb).wait()
                pltpu.make_async_copy(q_hbm.at[rvb], qb, ssb).wait()
                tcompute(gb, qb, (w0 + 1) * WA)
            _scale_window(bufb, tbuf, (w0 + 1) * WA, WA)
            pltpu.sync_copy(bufb, acc.at[rvb], add=True)

            @pl.when(i < WINA // 2 - 1)
            def _():
                _unpack(pw, (w0 + 3) * WA, WA, cvb, rvb)
                pltpu.async_copy(z_hbm.at[cvb], bufb, sgb)
                if k == 0:
                    pltpu.async_copy(g_hbm.at[cvb], gb, ssb)
                    pltpu.async_copy(q_hbm.at[rvb], qb, ssb)
            return 0
        lax.fori_loop(0, WINA // 2, win2, 0)
        plsc.subcore_barrier()
        pltpu.sync_copy(acc.at[pl.ds(s * RPT, RPT)],
                        out_hbm.at[k, c, pl.ds(s * RPT, RPT)])


def _scale_body(x_ref, w_ref, d0_ref, d1_ref, o_ref):
    xw = jnp.dot(x_ref[...], w_ref[...], preferred_element_type=f32)
    dinv = lax.rsqrt(1.0 + d0_ref[...] + d1_ref[...])
    o_ref[...] = xw * dinv


_scale = pl.pallas_call(
    _scale_body,
    grid=(GRID,),
    in_specs=[
        pl.BlockSpec((RB, F), lambda i: (i, 0)),
        pl.BlockSpec((F, F), lambda i: (0, 0)),
        pl.BlockSpec((RB, 1), lambda i: (i, 0)),
        pl.BlockSpec((RB, 1), lambda i: (i, 0)),
    ],
    out_specs=pl.BlockSpec((RB, F), lambda i: (i, 0)),
    out_shape=jax.ShapeDtypeStruct((NPAD, F), f32),
)


def _mid_body(hs0_ref, hs1_ref, xws_ref, d0_ref, d1_ref, x_ref, b1_ref,
              wfca_ref, wfcb_ref, bfcp_ref, qb_ref,
              g_ref, q_ref, z0_ref, z1_ref, z2_ref, z3_ref):
    dinv = lax.rsqrt(1.0 + d0_ref[...] + d1_ref[...])
    h = (hs0_ref[...] + hs1_ref[...] + xws_ref[...]) * dinv + b1_ref[...]
    lc = jnp.dot(h, wfca_ref[...], preferred_element_type=f32) + bfcp_ref[...]
    lr = jnp.dot(h, wfcb_ref[...], preferred_element_type=f32) + qb_ref[...]
    g = jnp.exp(lc - jnp.max(lc, axis=1, keepdims=True))
    q = jnp.exp(lr - jnp.max(lr, axis=1, keepdims=True))
    g_ref[...] = g
    q_ref[...] = q
    xb = x_ref[...]
    z0_ref[...] = xb * g[:, 0:1]
    z1_ref[...] = xb * g[:, 1:2]
    z2_ref[...] = xb * g[:, 2:3]
    z3_ref[...] = xb * g[:, 3:4]


_mid = pl.pallas_call(
    _mid_body,
    grid=(GRID,),
    in_specs=[
        pl.BlockSpec((RB, F), lambda i: (i, 0)),
        pl.BlockSpec((RB, F), lambda i: (i, 0)),
        pl.BlockSpec((RB, F), lambda i: (i, 0)),
        pl.BlockSpec((RB, 1), lambda i: (i, 0)),
        pl.BlockSpec((RB, 1), lambda i: (i, 0)),
        pl.BlockSpec((RB, F), lambda i: (i, 0)),
        pl.BlockSpec((1, F), lambda i: (0, 0)),
        pl.BlockSpec((F, 16), lambda i: (0, 0)),
        pl.BlockSpec((F, 16), lambda i: (0, 0)),
        pl.BlockSpec((1, 16), lambda i: (0, 0)),
        pl.BlockSpec((1, 16), lambda i: (0, 0)),
    ],
    out_specs=[
        pl.BlockSpec((RB, 16), lambda i: (i, 0)),
        pl.BlockSpec((RB, 16), lambda i: (i, 0)),
        pl.BlockSpec((RB, F), lambda i: (i, 0)),
        pl.BlockSpec((RB, F), lambda i: (i, 0)),
        pl.BlockSpec((RB, F), lambda i: (i, 0)),
        pl.BlockSpec((RB, F), lambda i: (i, 0)),
    ],
    out_shape=[
        jax.ShapeDtypeStruct((NPAD, 16), f32),
        jax.ShapeDtypeStruct((NPAD, 16), f32),
        jax.ShapeDtypeStruct((NPAD, F), f32),
        jax.ShapeDtypeStruct((NPAD, F), f32),
        jax.ShapeDtypeStruct((NPAD, F), f32),
        jax.ShapeDtypeStruct((NPAD, F), f32),
    ],
)


def _final_body(ag_ref, q_ref, w_ref, b_ref, oa_ref):
    k = pl.program_id(0)
    a = ag_ref[0, 0] + ag_ref[0, 1]
    qall = q_ref[...]
    onehot = lax.broadcasted_iota(i32, (RB, 16), 1) == k
    qk = jnp.sum(jnp.where(onehot, qall, 0.0), axis=1, keepdims=True)
    o = jnp.dot(a * qk, w_ref[0], preferred_element_type=f32) + b_ref[0]
    nrm = jnp.sqrt(jnp.sum(o * o, axis=1, keepdims=True))
    o = o / jnp.maximum(nrm, 1e-12)
    oa_ref[0] = o


_final = pl.pallas_call(
    _final_body,
    grid=(CH, GRID),
    in_specs=[
        pl.BlockSpec((1, NC, RB, F), lambda k, i: (k, 0, i, 0)),
        pl.BlockSpec((RB, 16), lambda k, i: (i, 0)),
        pl.BlockSpec((1, F, F), lambda k, i: (k, 0, 0)),
        pl.BlockSpec((1, 1, F), lambda k, i: (k, 0, 0)),
    ],
    out_specs=pl.BlockSpec((1, RB, F), lambda k, i: (k, i, 0)),
    out_shape=jax.ShapeDtypeStruct((CH, NPAD, F), f32),
)


def kernel(x, edge_index, W1, b1, Wfc, bfc, Wconv, bias_list):
    row = edge_index[0].astype(i32)
    col = edge_index[1].astype(i32)
    padi = (N + (jnp.arange(EP - E, dtype=i32) % (NPAD - N))).astype(i32)
    rowp = jnp.concatenate([row, padi])
    colp = jnp.concatenate([col, padi])
    packed = (colp | (rowp << 16)).reshape(NW, EPW)
    x_p = jnp.pad(x, ((0, NPAD - N), (0, 0)))

    degp = _deg_sc(packed)
    d0 = degp[0].reshape(NPAD, 1)
    d1 = degp[1].reshape(NPAD, 1)
    xws = _scale(x_p, W1, d0, d1)
    hsump = _hsum_sc(packed, xws)

    wfca = jnp.pad(Wfc[:F], ((0, 0), (0, 12)))
    wfcb = jnp.pad(Wfc[F:], ((0, 0), (0, 12)))
    neg = jnp.full((12,), -1e30, f32)
    bfcp = jnp.concatenate([bfc, neg]).reshape(1, 16)
    qb = jnp.concatenate([jnp.zeros((CH,), f32), neg]).reshape(1, 16)
    g, q, z0, z1, z2, z3 = _mid(
        hsump[0], hsump[1], xws, d0, d1, x_p, b1.reshape(1, F),
        wfca, wfcb, bfcp, qb)

    aggp = _agg_sc(packed, g, q, z0, z1, z2, z3)
    oa = _final(aggp, q, Wconv, bias_list.reshape(CH, 1, F))

    output = oa[:, :N].transpose(1, 0, 2).reshape(N, CH * F)
    xs = oa[:2, :N].reshape(2 * N, F)
    xus = oa[2:, :N].reshape(2 * N, F)
    return (output, xs, xus)
